# Initial kernel scaffold; baseline (speedup 1.0000x reference)
import jax
import jax.numpy as jnp
from jax.experimental import pallas as pl


def kernel(model):
    raise NotImplementedError("write your pallas kernel here")


# speedup vs baseline: 9.3835x; 9.3835x over previous
import functools

import jax
import jax.numpy as jnp
from jax import lax
from jax.experimental import pallas as pl
from jax.experimental.pallas import tpu as pltpu
from jax.experimental.pallas import tpu_sc as plsc

P = 4096
D = 4096
K = 8
NC = 2
NS = 16
L = 16
NW = NC * NS
ROWS_PER_W = P // NW
R_TILE = 8
CHUNKS = D // L


def _top8_sum(ms):
    lane_iota = lax.iota(jnp.int32, L)
    total = jnp.float32(0.0)
    ms = list(ms)
    for _ in range(K):
        head = ms[0]
        m = jnp.max(head)
        total = total + m
        first = plsc.all_reduce_ffs(head == m)
        lane = lane_iota == first
        for i in range(K - 1):
            ms[i] = jnp.where(lane, ms[i + 1], ms[i])
        ms[K - 1] = jnp.where(lane, jnp.zeros((L,), jnp.float32), ms[K - 1])
    return total


def _row_result(buf, r):
    zero = jnp.zeros((L,), jnp.float32)

    def body(j, carry):
        s, m0, m1, m2, m3, m4, m5, m6, m7 = carry
        v = jnp.abs(buf[r, pl.ds(j * L, L)])
        s = s + v
        hi = jnp.maximum(m0, v); v = jnp.minimum(m0, v); m0 = hi
        hi = jnp.maximum(m1, v); v = jnp.minimum(m1, v); m1 = hi
        hi = jnp.maximum(m2, v); v = jnp.minimum(m2, v); m2 = hi
        hi = jnp.maximum(m3, v); v = jnp.minimum(m3, v); m3 = hi
        hi = jnp.maximum(m4, v); v = jnp.minimum(m4, v); m4 = hi
        hi = jnp.maximum(m5, v); v = jnp.minimum(m5, v); m5 = hi
        hi = jnp.maximum(m6, v); v = jnp.minimum(m6, v); m6 = hi
        hi = jnp.maximum(m7, v); m7 = hi
        return (s, m0, m1, m2, m3, m4, m5, m6, m7)

    init = (zero,) * 9
    s, m0, m1, m2, m3, m4, m5, m6, m7 = lax.fori_loop(
        0, CHUNKS, body, init, unroll=4)

    top8_sum = _top8_sum([m0, m1, m2, m3, m4, m5, m6, m7])
    row_sum = jnp.sum(s)
    return top8_sum * (1.0 / K) - row_sum * (1.0 / D)


def _sc_kernel(x_hbm, out_hbm, buf, acc_vmem):
    wid = lax.axis_index("s") * NC + lax.axis_index("c")
    base = wid * ROWS_PER_W

    def tile_body(t, acc):
        pltpu.sync_copy(x_hbm.at[pl.ds(base + t * R_TILE, R_TILE), :], buf)
        for r in range(R_TILE):
            acc = acc + _row_result(buf, r)
        return acc

    acc = lax.fori_loop(0, ROWS_PER_W // R_TILE, tile_body,
                        jnp.zeros((L,), jnp.float32))
    acc_vmem[...] = acc
    pltpu.sync_copy(acc_vmem, out_hbm.at[wid])


@jax.jit
def _run(x):
    mesh = plsc.VectorSubcoreMesh(core_axis_name="c", subcore_axis_name="s")
    partials = pl.kernel(
        _sc_kernel,
        out_type=jax.ShapeDtypeStruct((NW, L), jnp.float32),
        mesh=mesh,
        scratch_types=[
            pltpu.VMEM((R_TILE, D), jnp.float32),
            pltpu.VMEM((L,), jnp.float32),
        ],
        compiler_params=pltpu.CompilerParams(needs_layout_passes=False),
    )(x)
    return jnp.sum(partials[:, 0]) * (1.0 / P)


def kernel(model):
    x = model.reshape(P, D)
    return _run(x)


# speedup vs baseline: 11.2849x; 1.2026x over previous
import functools

import jax
import jax.numpy as jnp
from jax import lax
from jax.experimental import pallas as pl
from jax.experimental.pallas import tpu as pltpu
from jax.experimental.pallas import tpu_sc as plsc

P = 4096
D = 4096
K = 8
NC = 2
NS = 16
L = 16
NW = NC * NS
ROWS_PER_W = P // NW
R_TILE = 8
CHUNKS = D // L


def _top8_sum(ms):
    lane_iota = lax.iota(jnp.int32, L)
    total = jnp.float32(0.0)
    ms = list(ms)
    for _ in range(K):
        head = ms[0]
        m = jnp.max(head)
        total = total + m
        first = plsc.all_reduce_ffs(head == m)
        lane = lane_iota == first
        for i in range(K - 1):
            ms[i] = jnp.where(lane, ms[i + 1], ms[i])
        ms[K - 1] = jnp.where(lane, jnp.zeros((L,), jnp.float32), ms[K - 1])
    return total


_SORT8 = ((0, 1), (2, 3), (4, 5), (6, 7),
          (0, 2), (1, 3), (4, 6), (5, 7),
          (1, 2), (5, 6),
          (0, 4), (1, 5), (2, 6), (3, 7),
          (2, 4), (3, 5),
          (1, 2), (3, 4), (5, 6))


def _sort8_desc(vs):
    vs = list(vs)
    for a, b in _SORT8:
        hi = jnp.maximum(vs[a], vs[b])
        lo = jnp.minimum(vs[a], vs[b])
        vs[a], vs[b] = hi, lo
    return vs


def _merge_top8(ms, bs):
    c = [jnp.maximum(ms[i], bs[7 - i]) for i in range(8)]
    for dist in (4, 2, 1):
        for base in range(0, 8, 2 * dist):
            for i in range(base, base + dist):
                hi = jnp.maximum(c[i], c[i + dist])
                lo = jnp.minimum(c[i], c[i + dist])
                c[i], c[i + dist] = hi, lo
    return c


def _row_result(buf, r):
    zero = jnp.zeros((L,), jnp.float32)

    def body(j, carry):
        s = carry[0]
        ms = list(carry[1:])
        base = j * (8 * L)
        vs = []
        for t in range(8):
            v = jnp.abs(buf[r, pl.ds(base + t * L, L)])
            s = s + v
            vs.append(v)
        bs = _sort8_desc(vs)
        ms = _merge_top8(ms, bs)
        return (s, *ms)

    init = (zero,) * 9
    out = lax.fori_loop(0, CHUNKS // 8, body, init, unroll=2)
    s = out[0]
    ms = list(out[1:])

    top8_sum = _top8_sum(ms)
    row_sum = jnp.sum(s)
    return top8_sum * (1.0 / K) - row_sum * (1.0 / D)


def _sc_kernel(x_hbm, out_hbm, buf, acc_vmem):
    wid = lax.axis_index("s") * NC + lax.axis_index("c")
    base = wid * ROWS_PER_W

    def tile_body(t, acc):
        pltpu.sync_copy(x_hbm.at[pl.ds(base + t * R_TILE, R_TILE), :], buf)
        for r in range(R_TILE):
            acc = acc + _row_result(buf, r)
        return acc

    acc = lax.fori_loop(0, ROWS_PER_W // R_TILE, tile_body,
                        jnp.zeros((L,), jnp.float32))
    acc_vmem[...] = acc
    pltpu.sync_copy(acc_vmem, out_hbm.at[wid])


@jax.jit
def _run(x):
    mesh = plsc.VectorSubcoreMesh(core_axis_name="c", subcore_axis_name="s")
    partials = pl.kernel(
        _sc_kernel,
        out_type=jax.ShapeDtypeStruct((NW, L), jnp.float32),
        mesh=mesh,
        scratch_types=[
            pltpu.VMEM((R_TILE, D), jnp.float32),
            pltpu.VMEM((L,), jnp.float32),
        ],
        compiler_params=pltpu.CompilerParams(needs_layout_passes=False),
    )(x)
    return jnp.sum(partials[:, 0]) * (1.0 / P)


def kernel(model):
    x = model.reshape(P, D)
    return _run(x)


# speedup vs baseline: 14.8130x; 1.3126x over previous
import functools

import jax
import jax.numpy as jnp
from jax import lax
from jax.experimental import pallas as pl
from jax.experimental.pallas import tpu as pltpu
from jax.experimental.pallas import tpu_sc as plsc

P = 4096
D = 4096
K = 8
NC = 2
NS = 16
L = 16
NW = NC * NS
ROWS_PER_W = P // NW
R_TILE = 8
CHUNKS = D // L


def _top8_sum(ms):
    lane_iota = lax.iota(jnp.int32, L)
    total = jnp.float32(0.0)
    ms = list(ms)
    for _ in range(K):
        head = ms[0]
        m = jnp.max(head)
        total = total + m
        first = plsc.all_reduce_ffs(head == m)
        lane = lane_iota == first
        for i in range(K - 1):
            ms[i] = jnp.where(lane, ms[i + 1], ms[i])
        ms[K - 1] = jnp.where(lane, jnp.zeros((L,), jnp.float32), ms[K - 1])
    return total


_SORT8 = ((0, 1), (2, 3), (4, 5), (6, 7),
          (0, 2), (1, 3), (4, 6), (5, 7),
          (1, 2), (5, 6),
          (0, 4), (1, 5), (2, 6), (3, 7),
          (2, 4), (3, 5),
          (1, 2), (3, 4), (5, 6))


def _sort8_desc(vs):
    vs = list(vs)
    for a, b in _SORT8:
        hi = jnp.maximum(vs[a], vs[b])
        lo = jnp.minimum(vs[a], vs[b])
        vs[a], vs[b] = hi, lo
    return vs


def _merge_top8(ms, bs):
    c = [jnp.maximum(ms[i], bs[7 - i]) for i in range(8)]
    for dist in (4, 2, 1):
        for base in range(0, 8, 2 * dist):
            for i in range(base, base + dist):
                hi = jnp.maximum(c[i], c[i + dist])
                lo = jnp.minimum(c[i], c[i + dist])
                c[i], c[i + dist] = hi, lo
    return c


def _sort_asc(v):
    return plsc.sort_key_val(v, v)[0]


def _sort_desc(v):
    return plsc.sort_key_val(v, v, descending=True)[0]


def _tile_result(buf):
    zero = jnp.zeros((L,), jnp.float32)

    def body(j, carry):
        ss = list(carry[:R_TILE])
        cs = list(carry[R_TILE:])
        for r in range(R_TILE):
            v = jnp.abs(buf[r, pl.ds(j * L, L)])
            ss[r] = ss[r] + v
            cs[r] = _sort_desc(jnp.maximum(cs[r], _sort_asc(v)))
        return (*ss, *cs)

    init = (zero,) * (2 * R_TILE)
    out = lax.fori_loop(0, CHUNKS, body, init, unroll=2)
    ss = out[:R_TILE]
    cs = out[R_TILE:]

    keep = lax.iota(jnp.int32, L) < K
    total = jnp.float32(0.0)
    for r in range(R_TILE):
        top8_sum = jnp.sum(jnp.where(keep, cs[r], zero))
        row_sum = jnp.sum(ss[r])
        total = total + (top8_sum * (1.0 / K) - row_sum * (1.0 / D))
    return total


def _sc_kernel(x_hbm, out_hbm, buf, acc_vmem):
    wid = lax.axis_index("s") * NC + lax.axis_index("c")
    base = wid * ROWS_PER_W

    def tile_body(t, acc):
        pltpu.sync_copy(x_hbm.at[pl.ds(base + t * R_TILE, R_TILE), :], buf)
        return acc + _tile_result(buf)

    acc = lax.fori_loop(0, ROWS_PER_W // R_TILE, tile_body,
                        jnp.float32(0.0))
    acc_vmem[...] = jnp.zeros((L,), jnp.float32) + acc
    pltpu.sync_copy(acc_vmem, out_hbm.at[wid])


@jax.jit
def _run(x):
    mesh = plsc.VectorSubcoreMesh(core_axis_name="c", subcore_axis_name="s")
    partials = pl.kernel(
        _sc_kernel,
        out_type=jax.ShapeDtypeStruct((NW, L), jnp.float32),
        mesh=mesh,
        scratch_types=[
            pltpu.VMEM((R_TILE, D), jnp.float32),
            pltpu.VMEM((L,), jnp.float32),
        ],
        compiler_params=pltpu.CompilerParams(needs_layout_passes=False),
    )(x)
    return jnp.sum(partials[:, 0]) * (1.0 / P)


def kernel(model):
    x = model.reshape(P, D)
    return _run(x)
